# baseline (speedup 1.0000x reference)
import functools

import jax
import jax.numpy as jnp
import numpy as np
from jax import lax
from jax.experimental import pallas as pl
from jax.experimental.pallas import tpu as pltpu
from jax.experimental.pallas import tpu_sc as plsc

B = 16384
F = 26
D = 128
ND = 13
VOC = 100000
H0, H1, H2 = 1024, 512, 256
NW = 32
CHUNK = 128
BM = 1024
NSPLIT = 4
BS = B // NSPLIT
NBUF = 3


def _gather_body(nch, idxv_hbm, idxw_hbm, v_hbm, w_hbm, outv_hbm, outws_hbm,
                 idxv_all, idxw_all, vbuf, wall, wsum, sem_v, sem_w):
    per_w = nch * CHUNK
    rows = per_w // F
    wid = lax.axis_index("s") * 2 + lax.axis_index("c")
    base = wid * per_w
    pltpu.sync_copy(idxv_hbm.at[wid], idxv_all)
    pltpu.sync_copy(idxw_hbm.at[wid], idxw_all)

    def fire(c):
        pltpu.async_copy(v_hbm.at[idxv_all.at[c]], vbuf.at[c % NBUF], sem_v)
        pltpu.async_copy(w_hbm.at[idxw_all.at[c]],
                         wall.at[pl.ds(c * CHUNK, CHUNK)], sem_w)

    def drain_writeback(c):
        pltpu.make_async_copy(
            v_hbm.at[idxv_all.at[c]], vbuf.at[c % NBUF], sem_v).wait()
        pltpu.sync_copy(vbuf.at[c % NBUF],
                        outv_hbm.at[pl.ds(base + c * CHUNK, CHUNK)])

    for c in range(NBUF - 1):
        fire(c)

    def body(c, carry):
        fire(c)
        drain_writeback(c - (NBUF - 1))
        return carry

    lax.fori_loop(NBUF - 1, nch, body, 0)
    for c in range(nch - NBUF + 1, nch):
        drain_writeback(c)

    pltpu.make_async_copy(w_hbm.at[pl.ds(0, per_w)], wall, sem_w).wait()
    for g in range(rows // 16):
        acc = wall[pl.ds(g * 16, 16)]
        for f in range(1, F):
            acc = acc + wall[pl.ds(f * rows + g * 16, 16)]
        wsum[pl.ds(g * 16, 16)] = acc
    pltpu.sync_copy(wsum, outws_hbm.at[pl.ds(wid * rows, rows)])


def _sc_gather(idxv, idxw, v_table, w_table, n):
    nf = n * F
    nch = nf // (NW * CHUNK)
    per_w = nch * CHUNK
    rows = per_w // F
    mesh = plsc.VectorSubcoreMesh(core_axis_name="c", subcore_axis_name="s")
    k = functools.partial(
        pl.kernel,
        mesh=mesh,
        out_type=[
            jax.ShapeDtypeStruct((nf, D), jnp.float32),
            jax.ShapeDtypeStruct((n,), jnp.float32),
        ],
        scratch_types=[
            pltpu.VMEM((nch, CHUNK), jnp.int32),
            pltpu.VMEM((nch, CHUNK), jnp.int32),
            pltpu.VMEM((NBUF, CHUNK, D), jnp.float32),
            pltpu.VMEM((per_w,), jnp.float32),
            pltpu.VMEM((rows,), jnp.float32),
            pltpu.SemaphoreType.DMA,
            pltpu.SemaphoreType.DMA,
        ],
    )(functools.partial(_gather_body, nch))
    return k(idxv, idxw, v_table, w_table)


def _mlp_body(scal, xv, dd, vdwt, wdwt, w0v, w0d, b0, w1t, b1,
              w2t, b2, cw1, out):
    d = dd[...]
    s = jnp.dot(d, vdwt[...], preferred_element_type=jnp.float32)
    sq = s * s
    xfs = []
    for f in range(F):
        xf = xv[f]
        s = s + xf
        sq = sq + xf * xf
        xfs.append(xf.astype(jnp.bfloat16))
    xb = jnp.concatenate(xfs, axis=1)
    h = (jnp.dot(xb, w0v[...], preferred_element_type=jnp.float32)
         + jnp.dot(d, w0d[...], preferred_element_type=jnp.float32) + b0[...])
    fm2 = 0.5 * jnp.sum(s * s - sq, axis=1, keepdims=True)
    fm1 = jnp.dot(d, wdwt[...], preferred_element_type=jnp.float32)
    h = jnp.maximum(h, 0.0)
    h = jnp.dot(h.astype(jnp.bfloat16), w1t[...],
                preferred_element_type=jnp.float32) + b1[...]
    h = jnp.maximum(h, 0.0)
    h = jnp.dot(h.astype(jnp.bfloat16), w2t[...],
                preferred_element_type=jnp.float32) + b2[...]
    h = jnp.maximum(h, 0.0)
    sc = scal[...]
    out[...] = (sc[:, 0:1] * (fm1 + fm2)
                + jnp.dot(h, cw1[...], preferred_element_type=jnp.float32)
                + sc[:, 1:2])


def _mlp_call(k, scal, xv, dense, vdwt, wdwt, w0v, w0d, b0, w1t,
              b1, w2t, b2, cw1, interpret=False):
    n = xv.shape[1]
    off = k * (n // BM)
    whole = lambda s: pl.BlockSpec(s, lambda i: (0,) * len(s))
    return pl.pallas_call(
        _mlp_body,
        grid=(n // BM,),
        in_specs=[
            whole((1, 2)),
            pl.BlockSpec((F, BM, D), lambda i: (0, i, 0)),
            pl.BlockSpec((BM, ND), lambda i: (i + off, 0)),
            whole((ND, D)),
            whole((ND, 1)),
            whole((F * D, H0)),
            whole((ND, H0)),
            whole((1, H0)),
            whole((H0, H1)),
            whole((1, H1)),
            whole((H1, H2)),
            whole((1, H2)),
            whole((H2, 1)),
        ],
        out_specs=pl.BlockSpec((BM, 1), lambda i: (i, 0)),
        out_shape=jax.ShapeDtypeStruct((n, 1), jnp.float32),
        interpret=interpret,
    )(scal, xv, dense, vdwt, wdwt, w0v, w0d, b0, w1t, b1, w2t, b2, cw1)


def kernel(sparse_features, dense_features, W0, W_sparse, W_dense_w, W_dense_b,
           V_sparse, V_dense_w, mlp_w0, mlp_b0, mlp_w1, mlp_b1, mlp_w2, mlp_b2,
           clf_w, clf_b):
    w1d = W_sparse.reshape(-1)
    cw00 = clf_w[0, 0]
    scal = jnp.stack(
        [cw00, cw00 * (W0[0] + W_dense_b[0]) + clf_b[0]]).reshape(1, 2)
    bf = jnp.bfloat16
    wargs = (
        V_dense_w.T, W_dense_w.T,
        mlp_w0[:, :F * D].T.astype(bf), mlp_w0[:, F * D:].T,
        mlp_b0.reshape(1, H0),
        mlp_w1.T.astype(bf), mlp_b1.reshape(1, H1),
        mlp_w2.T.astype(bf), mlp_b2.reshape(1, H2),
        clf_w[0, 1:].reshape(H2, 1))
    nch = BS * F // (NW * CHUNK)
    sf = sparse_features.astype(jnp.int32)
    gathered = []
    for k in range(NSPLIT):
        sfk = lax.slice_in_dim(sf, k * BS, (k + 1) * BS)
        idxv = sfk.T.reshape(NW, nch, CHUNK)
        rows_w = BS // NW
        idxw = sfk.reshape(NW, rows_w, F).transpose(0, 2, 1).reshape(
            NW, nch, CHUNK)
        gathered.append(_sc_gather(idxv, idxw, V_sparse, w1d, BS))
    outs = []
    for k in range(NSPLIT):
        outv, _ = gathered[k]
        outs.append(_mlp_call(k, scal, outv.reshape(F, BS, D),
                              dense_features, *wargs))
    ws_all = jnp.concatenate([g[1] for g in gathered]).reshape(B, 1)
    return jnp.concatenate(outs, axis=0) + cw00 * ws_all


# speedup vs baseline: 1.0360x; 1.0360x over previous
import functools

import jax
import jax.numpy as jnp
import numpy as np
from jax import lax
from jax.experimental import pallas as pl
from jax.experimental.pallas import tpu as pltpu
from jax.experimental.pallas import tpu_sc as plsc

B = 16384
F = 26
D = 128
ND = 13
VOC = 100000
H0, H1, H2 = 1024, 512, 256
NW = 32
CHUNK = 128
BM = 512
NSPLIT = 4
BS = B // NSPLIT
NBUF = 3


def _gather_body(nch, idxv_hbm, idxw_hbm, v_hbm, w_hbm, outv_hbm, outws_hbm,
                 idxv_all, idxw_all, vbuf, wall, wsum, sem_v, sem_w):
    per_w = nch * CHUNK
    rows = per_w // F
    wid = lax.axis_index("s") * 2 + lax.axis_index("c")
    base = wid * per_w
    pltpu.sync_copy(idxv_hbm.at[wid], idxv_all)
    pltpu.sync_copy(idxw_hbm.at[wid], idxw_all)

    def fire(c):
        pltpu.async_copy(v_hbm.at[idxv_all.at[c]], vbuf.at[c % NBUF], sem_v)
        pltpu.async_copy(w_hbm.at[idxw_all.at[c]],
                         wall.at[pl.ds(c * CHUNK, CHUNK)], sem_w)

    def drain_writeback(c):
        pltpu.make_async_copy(
            v_hbm.at[idxv_all.at[c]], vbuf.at[c % NBUF], sem_v).wait()
        pltpu.sync_copy(vbuf.at[c % NBUF],
                        outv_hbm.at[pl.ds(base + c * CHUNK, CHUNK)])

    for c in range(NBUF - 1):
        fire(c)

    def body(c, carry):
        fire(c)
        drain_writeback(c - (NBUF - 1))
        return carry

    lax.fori_loop(NBUF - 1, nch, body, 0)
    for c in range(nch - NBUF + 1, nch):
        drain_writeback(c)

    pltpu.make_async_copy(w_hbm.at[pl.ds(0, per_w)], wall, sem_w).wait()
    for g in range(rows // 16):
        acc = wall[pl.ds(g * 16, 16)]
        for f in range(1, F):
            acc = acc + wall[pl.ds(f * rows + g * 16, 16)]
        wsum[pl.ds(g * 16, 16)] = acc
    pltpu.sync_copy(wsum, outws_hbm.at[pl.ds(wid * rows, rows)])


def _sc_gather(idxv, idxw, v_table, w_table, n):
    nf = n * F
    nch = nf // (NW * CHUNK)
    per_w = nch * CHUNK
    rows = per_w // F
    mesh = plsc.VectorSubcoreMesh(core_axis_name="c", subcore_axis_name="s")
    k = functools.partial(
        pl.kernel,
        mesh=mesh,
        out_type=[
            jax.ShapeDtypeStruct((nf, D), jnp.float32),
            jax.ShapeDtypeStruct((n,), jnp.float32),
        ],
        scratch_types=[
            pltpu.VMEM((nch, CHUNK), jnp.int32),
            pltpu.VMEM((nch, CHUNK), jnp.int32),
            pltpu.VMEM((NBUF, CHUNK, D), jnp.float32),
            pltpu.VMEM((per_w,), jnp.float32),
            pltpu.VMEM((rows,), jnp.float32),
            pltpu.SemaphoreType.DMA,
            pltpu.SemaphoreType.DMA,
        ],
    )(functools.partial(_gather_body, nch))
    return k(idxv, idxw, v_table, w_table)


def _mlp_body(scal, xv, dd, vdwt, wdwt, w0v, w0d, b0, w1t, b1,
              w2t, b2, cw1, out):
    d = dd[...]
    s = jnp.dot(d, vdwt[...], preferred_element_type=jnp.float32)
    sq = s * s
    xfs = []
    for f in range(F):
        xf = xv[f]
        s = s + xf
        sq = sq + xf * xf
        xfs.append(xf.astype(jnp.bfloat16))
    xb = jnp.concatenate(xfs, axis=1)
    h = (jnp.dot(xb, w0v[...], preferred_element_type=jnp.float32)
         + jnp.dot(d, w0d[...], preferred_element_type=jnp.float32) + b0[...])
    fm2 = 0.5 * jnp.sum(s * s - sq, axis=1, keepdims=True)
    fm1 = jnp.dot(d, wdwt[...], preferred_element_type=jnp.float32)
    h = jnp.maximum(h, 0.0)
    h = jnp.dot(h.astype(jnp.bfloat16), w1t[...],
                preferred_element_type=jnp.float32) + b1[...]
    h = jnp.maximum(h, 0.0)
    h = jnp.dot(h.astype(jnp.bfloat16), w2t[...],
                preferred_element_type=jnp.float32) + b2[...]
    h = jnp.maximum(h, 0.0)
    sc = scal[...]
    out[...] = (sc[:, 0:1] * (fm1 + fm2)
                + jnp.dot(h, cw1[...], preferred_element_type=jnp.float32)
                + sc[:, 1:2])


def _mlp_call(k, scal, xv, dense, vdwt, wdwt, w0v, w0d, b0, w1t,
              b1, w2t, b2, cw1, interpret=False):
    n = xv.shape[1]
    off = k * (n // BM)
    whole = lambda s: pl.BlockSpec(s, lambda i: (0,) * len(s))
    return pl.pallas_call(
        _mlp_body,
        grid=(n // BM,),
        in_specs=[
            whole((1, 2)),
            pl.BlockSpec((F, BM, D), lambda i: (0, i, 0)),
            pl.BlockSpec((BM, ND), lambda i: (i + off, 0)),
            whole((ND, D)),
            whole((ND, 1)),
            whole((F * D, H0)),
            whole((ND, H0)),
            whole((1, H0)),
            whole((H0, H1)),
            whole((1, H1)),
            whole((H1, H2)),
            whole((1, H2)),
            whole((H2, 1)),
        ],
        out_specs=pl.BlockSpec((BM, 1), lambda i: (i, 0)),
        out_shape=jax.ShapeDtypeStruct((n, 1), jnp.float32),
        interpret=interpret,
    )(scal, xv, dense, vdwt, wdwt, w0v, w0d, b0, w1t, b1, w2t, b2, cw1)


def kernel(sparse_features, dense_features, W0, W_sparse, W_dense_w, W_dense_b,
           V_sparse, V_dense_w, mlp_w0, mlp_b0, mlp_w1, mlp_b1, mlp_w2, mlp_b2,
           clf_w, clf_b):
    w1d = W_sparse.reshape(-1)
    cw00 = clf_w[0, 0]
    scal = jnp.stack(
        [cw00, cw00 * (W0[0] + W_dense_b[0]) + clf_b[0]]).reshape(1, 2)
    bf = jnp.bfloat16
    wargs = (
        V_dense_w.T, W_dense_w.T,
        mlp_w0[:, :F * D].T.astype(bf), mlp_w0[:, F * D:].T,
        mlp_b0.reshape(1, H0),
        mlp_w1.T.astype(bf), mlp_b1.reshape(1, H1),
        mlp_w2.T.astype(bf), mlp_b2.reshape(1, H2),
        clf_w[0, 1:].reshape(H2, 1))
    nch = BS * F // (NW * CHUNK)
    sf = sparse_features.astype(jnp.int32)
    gathered = []
    for k in range(NSPLIT):
        sfk = lax.slice_in_dim(sf, k * BS, (k + 1) * BS)
        idxv = sfk.T.reshape(NW, nch, CHUNK)
        rows_w = BS // NW
        idxw = sfk.reshape(NW, rows_w, F).transpose(0, 2, 1).reshape(
            NW, nch, CHUNK)
        gathered.append(_sc_gather(idxv, idxw, V_sparse, w1d, BS))
    outs = []
    for k in range(NSPLIT):
        outv, _ = gathered[k]
        outs.append(_mlp_call(k, scal, outv.reshape(F, BS, D),
                              dense_features, *wargs))
    ws_all = jnp.concatenate([g[1] for g in gathered]).reshape(B, 1)
    return jnp.concatenate(outs, axis=0) + cw00 * ws_all
